# Initial kernel scaffold; baseline (speedup 1.0000x reference)
import jax
import jax.numpy as jnp
from jax.experimental import pallas as pl


def kernel(src_input, embeddings_table):
    raise NotImplementedError("write your pallas kernel here")


# speedup vs baseline: 5.7566x; 5.7566x over previous
import functools

import jax
import jax.numpy as jnp
from jax import lax
from jax.experimental import pallas as pl
from jax.experimental.pallas import tpu as pltpu
from jax.experimental.pallas import tpu_sc as plsc


@functools.cache
def _build(V, D, B):
    info = plsc.get_sparse_core_info()
    NC, NS = info.num_cores, info.num_subcores
    NW = NC * NS
    assert B % NW == 0
    b_per_w = B // NW
    C = 128
    assert b_per_w % C == 0
    n_chunks = b_per_w // C

    mesh = plsc.VectorSubcoreMesh(core_axis_name="c", subcore_axis_name="s")

    @functools.partial(
        pl.kernel,
        out_type=jax.ShapeDtypeStruct((B, D), jnp.float32),
        mesh=mesh,
        scratch_types=[
            pltpu.VMEM((n_chunks, C), jnp.int32),
            pltpu.VMEM((C, D), jnp.float32),
            pltpu.SemaphoreType.DMA,
        ],
    )
    def gather_kernel(idx_hbm, table_hbm, out_hbm, idx_v, rows_v, sem):
        wid = lax.axis_index("s") * NC + lax.axis_index("c")
        base = wid * b_per_w
        pltpu.sync_copy(idx_hbm.at[wid], idx_v)

        def chunk(i, carry):
            pltpu.async_copy(table_hbm.at[idx_v.at[i]], rows_v, sem).wait()
            pltpu.sync_copy(rows_v, out_hbm.at[pl.ds(base + i * C, C)])
            return carry

        lax.fori_loop(0, n_chunks, chunk, 0)

    return gather_kernel, NW, n_chunks, C


def kernel(src_input, embeddings_table):
    BATCH, SEQ = src_input.shape
    V, D = embeddings_table.shape
    B = BATCH * SEQ
    gather_kernel, NW, n_chunks, C = _build(V, D, B)
    idx = src_input.reshape(NW, n_chunks, C).astype(jnp.int32)
    out = gather_kernel(idx, embeddings_table)
    return out.reshape(BATCH, SEQ, D)


# speedup vs baseline: 7.7537x; 1.3469x over previous
import functools

import jax
import jax.numpy as jnp
from jax import lax
from jax.experimental import pallas as pl
from jax.experimental.pallas import tpu as pltpu
from jax.experimental.pallas import tpu_sc as plsc


@functools.cache
def _build(V, D, B):
    info = plsc.get_sparse_core_info()
    NC, NS = info.num_cores, info.num_subcores
    NW = NC * NS
    assert B % NW == 0
    b_per_w = B // NW
    C = 128
    assert b_per_w % C == 0
    n_chunks = b_per_w // C

    NBUF = 5
    assert n_chunks % NBUF == 0
    n_turns = n_chunks // NBUF

    mesh = plsc.VectorSubcoreMesh(core_axis_name="c", subcore_axis_name="s")

    @functools.partial(
        pl.kernel,
        out_type=jax.ShapeDtypeStruct((B, D), jnp.float32),
        mesh=mesh,
        scratch_types=[
            pltpu.VMEM((n_chunks, C), jnp.int32),
            [pltpu.VMEM((C, D), jnp.float32) for _ in range(NBUF)],
            [pltpu.SemaphoreType.DMA for _ in range(NBUF)],
            [pltpu.SemaphoreType.DMA for _ in range(NBUF)],
        ],
    )
    def gather_kernel(idx_hbm, table_hbm, out_hbm, idx_v, rows, sem_in, sem_out):
        wid = lax.axis_index("s") * NC + lax.axis_index("c")
        base = wid * b_per_w
        pltpu.sync_copy(idx_hbm.at[wid], idx_v)

        for b in range(NBUF):
            pltpu.async_copy(table_hbm.at[idx_v.at[b]], rows[b], sem_in[b])

        def turn(j, carry):
            for b in range(NBUF):
                i = j * NBUF + b
                pltpu.make_async_copy(
                    table_hbm.at[idx_v.at[0]], rows[b], sem_in[b]
                ).wait()
                pltpu.async_copy(
                    rows[b], out_hbm.at[pl.ds(base + i * C, C)], sem_out[b]
                )
            for b in range(NBUF):
                i_next = (j + 1) * NBUF + b

                @pl.when(i_next < n_chunks)
                def _():
                    pltpu.make_async_copy(
                        rows[b], out_hbm.at[pl.ds(base, C)], sem_out[b]
                    ).wait()
                    pltpu.async_copy(
                        table_hbm.at[idx_v.at[i_next]], rows[b], sem_in[b]
                    )

            return carry

        lax.fori_loop(0, n_turns, turn, 0)
        for b in range(NBUF):
            pltpu.make_async_copy(
                rows[b], out_hbm.at[pl.ds(base, C)], sem_out[b]
            ).wait()

    return gather_kernel, NW, n_chunks, C


def kernel(src_input, embeddings_table):
    BATCH, SEQ = src_input.shape
    V, D = embeddings_table.shape
    B = BATCH * SEQ
    gather_kernel, NW, n_chunks, C = _build(V, D, B)
    idx = src_input.reshape(NW, n_chunks, C).astype(jnp.int32)
    out = gather_kernel(idx, embeddings_table)
    return out.reshape(BATCH, SEQ, D)
